# baseline (speedup 1.0000x reference)
import jax
import jax.numpy as jnp
from jax.experimental import pallas as pl

NUM_Q = 8
K = 1024
DIM = 64
COMMIT_W = 1.0
BLK = 1024


def _rvq_kernel(y_ref, cb_ref, yhat_ref, idx_ref, ssq_ref):
    i = pl.program_id(0)

    @pl.when(i == 0)
    def _init():
        ssq_ref[...] = jnp.zeros_like(ssq_ref)

    y = y_ref[...]
    blk = y.shape[0]
    res = y
    yhat = jnp.zeros_like(y)
    lane_iota = jax.lax.broadcasted_iota(jnp.int32, (blk, K), 1)
    q_iota = jax.lax.broadcasted_iota(jnp.int32, (1, NUM_Q), 1)
    idx_cols = []
    ssq_acc = jnp.zeros((1, NUM_Q), jnp.float32)
    for qi in range(NUM_Q):
        cb = cb_ref[qi]
        x2 = jnp.sum(res * res, axis=1, keepdims=True)
        c2 = jnp.sum(cb * cb, axis=1)[None, :]
        xc = jax.lax.dot_general(res.astype(jnp.bfloat16),
                                 cb.astype(jnp.bfloat16),
                                 (((1,), (1,)), ((), ())),
                                 preferred_element_type=jnp.float32)
        d = x2 - 2.0 * xc + c2
        dmin = jnp.min(d, axis=1, keepdims=True)
        idx = jnp.min(jnp.where(d == dmin, lane_iota, K),
                      axis=1, keepdims=True)
        onehot = (lane_iota == idx).astype(jnp.float32)
        q = jax.lax.dot_general(onehot, cb, (((1,), (0,)), ((), ())),
                                precision=jax.lax.Precision.HIGHEST,
                                preferred_element_type=jnp.float32)
        res = res - q
        yhat = yhat + q
        ssq_acc = ssq_acc + jnp.where(q_iota == qi,
                                      jnp.sum(res * res), 0.0)
        idx_cols.append(idx)
    yhat_ref[...] = yhat
    idx_ref[...] = jnp.concatenate(idx_cols, axis=1)
    ssq_ref[...] += ssq_acc


def kernel(y, codebooks):
    b, _ = y.shape
    grid = (b // BLK,)
    yhat, idx, ssq = pl.pallas_call(
        _rvq_kernel,
        grid=grid,
        in_specs=[
            pl.BlockSpec((BLK, DIM), lambda i: (i, 0)),
            pl.BlockSpec((NUM_Q, K, DIM), lambda i: (0, 0, 0)),
        ],
        out_specs=[
            pl.BlockSpec((BLK, DIM), lambda i: (i, 0)),
            pl.BlockSpec((BLK, NUM_Q), lambda i: (i, 0)),
            pl.BlockSpec((1, NUM_Q), lambda i: (0, 0)),
        ],
        out_shape=[
            jax.ShapeDtypeStruct((b, DIM), jnp.float32),
            jax.ShapeDtypeStruct((b, NUM_Q), jnp.int32),
            jax.ShapeDtypeStruct((1, NUM_Q), jnp.float32),
        ],
    )(y, codebooks)
    losses_per_layer = COMMIT_W * (ssq[0] / (b * DIM))
    loss_vq = jnp.mean(losses_per_layer)
    return yhat, idx, loss_vq, losses_per_layer


# speedup vs baseline: 1.8987x; 1.8987x over previous
import jax
import jax.numpy as jnp
from jax.experimental import pallas as pl

NUM_Q = 8
K = 1024
DIM = 64
COMMIT_W = 1.0
BLK = 1024


def _rvq_kernel(y_ref, cb_ref, yhat_ref, idx_ref, ssq_ref):
    i = pl.program_id(0)

    @pl.when(i == 0)
    def _init():
        ssq_ref[...] = jnp.zeros_like(ssq_ref)

    y = y_ref[...]
    blk = y.shape[0]
    res = y
    lane_iota = jax.lax.broadcasted_iota(jnp.int32, (blk, K), 1)
    q_iota = jax.lax.broadcasted_iota(jnp.int32, (1, NUM_Q), 1)
    idx_cols = []
    ssq_acc = jnp.zeros((1, NUM_Q), jnp.float32)
    mm = lambda a, b, dims: jax.lax.dot_general(
        a, b, (dims, ((), ())), preferred_element_type=jnp.float32)
    for qi in range(NUM_Q):
        cb = cb_ref[qi]
        s1 = cb.astype(jnp.bfloat16)
        r1 = cb - s1.astype(jnp.float32)
        s2 = r1.astype(jnp.bfloat16)
        s3 = (r1 - s2.astype(jnp.float32)).astype(jnp.bfloat16)
        x2 = jnp.sum(res * res, axis=1, keepdims=True)
        c2 = jnp.sum(cb * cb, axis=1)[None, :]
        xc = mm(res.astype(jnp.bfloat16), s1, ((1,), (1,)))
        d = x2 - 2.0 * xc + c2
        dmin = jnp.min(d, axis=1, keepdims=True)
        idx = jnp.min(jnp.where(d == dmin, lane_iota, K),
                      axis=1, keepdims=True)
        onehot = (lane_iota == idx).astype(jnp.bfloat16)
        q = ((mm(onehot, s1, ((1,), (0,))) + mm(onehot, s2, ((1,), (0,))))
             + mm(onehot, s3, ((1,), (0,))))
        res = res - q
        ssq_acc = ssq_acc + jnp.where(q_iota == qi,
                                      jnp.sum(res * res), 0.0)
        idx_cols.append(idx)
    yhat_ref[...] = y - res
    idx_ref[...] = jnp.concatenate(idx_cols, axis=1)
    ssq_ref[...] += ssq_acc


def kernel(y, codebooks):
    b, _ = y.shape
    grid = (b // BLK,)
    yhat, idx, ssq = pl.pallas_call(
        _rvq_kernel,
        grid=grid,
        in_specs=[
            pl.BlockSpec((BLK, DIM), lambda i: (i, 0)),
            pl.BlockSpec((NUM_Q, K, DIM), lambda i: (0, 0, 0)),
        ],
        out_specs=[
            pl.BlockSpec((BLK, DIM), lambda i: (i, 0)),
            pl.BlockSpec((BLK, NUM_Q), lambda i: (i, 0)),
            pl.BlockSpec((1, NUM_Q), lambda i: (0, 0)),
        ],
        out_shape=[
            jax.ShapeDtypeStruct((b, DIM), jnp.float32),
            jax.ShapeDtypeStruct((b, NUM_Q), jnp.int32),
            jax.ShapeDtypeStruct((1, NUM_Q), jnp.float32),
        ],
    )(y, codebooks)
    losses_per_layer = COMMIT_W * (ssq[0] / (b * DIM))
    loss_vq = jnp.mean(losses_per_layer)
    return yhat, idx, loss_vq, losses_per_layer
